# baseline (speedup 1.0000x reference)
import functools

import jax
import jax.numpy as jnp
import numpy as np
from jax import lax
from jax.experimental import pallas as pl
from jax.experimental.pallas import tpu as pltpu
from jax.experimental.pallas import tpu_sc as plsc

_D, _Z, _Y, _X = 64, 64, 64, 64
_NW = 32
_CH = 48
_G16 = _CH // 16
_NROW = 2 * _Z * (_Y // 2)

_H = np.array([[2., -2., 1., 1.], [-3., 3., -2., -1.],
               [0., 0., 1., 0.], [1., 0., 0., 0.]])
_CRB = _H @ np.array([[0., 1., 0., 0.], [0., 0., 1., 0.],
                      [-0.5, 0., 0.5, 0.], [0., -0.5, 0., 0.5]])


def _collapse_body(idp_ref, cd_ref, knots_ref, out_ref):
    d = pl.program_id(0)
    w = cd_ref[d]
    v = knots_ref[0]

    @pl.when(d == 0)
    def _():
        out_ref[...] = jnp.zeros_like(out_ref)

    out_ref[0] += w * v
    vr = jnp.concatenate([v[:, 1:, :], v[:, :1, :]], axis=1)
    out_ref[1] += w * vr


def _collapse(idp, cd, knots4):
    grid_spec = pltpu.PrefetchScalarGridSpec(
        num_scalar_prefetch=1,
        grid=(4,),
        in_specs=[
            pl.BlockSpec(memory_space=pltpu.SMEM),
            pl.BlockSpec((1, _Z, _Y, _X), lambda d, idp_ref: (idp_ref[d], 0, 0, 0)),
        ],
        out_specs=pl.BlockSpec((2, _Z, _Y, _X), lambda d, idp_ref: (0, 0, 0, 0)),
    )
    return pl.pallas_call(
        _collapse_body,
        grid_spec=grid_spec,
        out_shape=jax.ShapeDtypeStruct((2, _Z, _Y, _X), jnp.float32),
    )(idp, cd, knots4)


def _weights(s):
    w0 = ((-0.5 * s + 1.0) * s - 0.5) * s
    w1 = (1.5 * s - 2.5) * s * s + 1.0
    w2 = ((-1.5 * s + 2.0) * s + 0.5) * s
    w3 = (0.5 * s - 0.5) * s * s
    return w0, w1, w2, w3


def _axis_prep(coord, hi):
    i0 = coord.astype(jnp.int32)
    s = jnp.clip(coord - i0.astype(jnp.float32), 0.0, 1.0)
    b = jnp.clip(i0 - 1, 0, hi - 4)
    return b, _weights(s)


def _interp_body(pw, table, zc, yc, xc, out,
                 zbuf, ybuf, xbuf, idxbufs, valss, wbufs, xbbs, outbuf, sem):
    wid = lax.axis_index("s") * 2 + lax.axis_index("c")
    base = wid * pw
    pltpu.sync_copy(zc.at[pl.ds(base, pw + _CH)], zbuf)
    pltpu.sync_copy(yc.at[pl.ds(base, pw + _CH)], ybuf)
    pltpu.sync_copy(xc.at[pl.ds(base, pw + _CH)], xbuf)

    iota = lax.iota(jnp.int32, 16)

    def phase1(c, nb):
        idxbuf, wbuf, xbb = idxbufs[nb], wbufs[nb], xbbs[nb]
        for g in range(_G16):
            p0 = c * _CH + g * 16
            g16 = pl.ds(g * 16, 16)
            z = zbuf[pl.ds(p0, 16)]
            y = ybuf[pl.ds(p0, 16)]
            x = xbuf[pl.ds(p0, 16)]
            zb, wz = _axis_prep(z, _Z)
            yb, wy = _axis_prep(y, _Y)
            xb, wx = _axis_prep(x, _X)
            for k in range(4):
                wbuf[k, g16] = wz[k]
                wbuf[4 + k, g16] = wy[k]
                wbuf[8 + k, g16] = wx[k]
            xbb[g16] = xb
            ys = jnp.bitwise_and(yb, 1)
            k0 = lax.shift_right_logical(yb, 1)
            base0 = ys * (_Z * (_Y // 2)) + zb * (_Y // 2) + k0
            for a in range(4):
                for e in range(2):
                    idxbuf[a * 2 + e, g16] = base0 + (a * (_Y // 2) + e)
        for j in range(8):
            pltpu.async_copy(table.at[idxbuf.at[j]],
                             valss[nb].at[pl.ds(j * _CH, _CH)], sem)

    def drain(nb):
        for j in range(8):
            pltpu.make_async_copy(table.at[pl.ds(0, _CH)],
                                  valss[nb].at[pl.ds(j * _CH, _CH)], sem).wait()

    def phase2(c, nb):
        vals, wbuf, xbb = valss[nb], wbufs[nb], xbbs[nb]
        for g in range(_G16):
            g16 = pl.ds(g * 16, 16)
            pvec = iota + g * 16
            wz = [wbuf[k, g16] for k in range(4)]
            wy = [wbuf[4 + k, g16] for k in range(4)]
            wx = [wbuf[8 + k, g16] for k in range(4)]
            xb = xbb[g16]
            acc = jnp.zeros((16,), jnp.float32)
            for b in range(4):
                colv = xb + (b & 1) * 64
                for a in range(4):
                    rowv = pvec + (a * 2 + (b >> 1)) * _CH
                    t = plsc.load_gather(vals, [rowv, colv]) * wx[0]
                    t += plsc.load_gather(vals, [rowv, colv + 1]) * wx[1]
                    t += plsc.load_gather(vals, [rowv, colv + 2]) * wx[2]
                    t += plsc.load_gather(vals, [rowv, colv + 3]) * wx[3]
                    acc += t * (wz[a] * wy[b])
            outbuf[pl.ds(c * _CH + g * 16, 16)] = acc

    nch = pw // _CH
    phase1(0, 0)

    def pair(c2, carry):
        ca = 2 * c2
        phase1(ca + 1, 1)
        drain(0)
        phase2(ca, 0)
        phase1(ca + 2, 0)
        drain(1)
        phase2(ca + 1, 1)
        return carry

    lax.fori_loop(0, nch // 2, pair, 0)
    drain(0)
    pltpu.sync_copy(outbuf, out.at[pl.ds(base, pw)])


def _interp(table_rows, zc, yc, xc, npad, pw):
    mesh = plsc.VectorSubcoreMesh(core_axis_name="c", subcore_axis_name="s")
    return pl.kernel(
        functools.partial(_interp_body, pw),
        out_type=jax.ShapeDtypeStruct((npad,), jnp.float32),
        mesh=mesh,
        compiler_params=pltpu.CompilerParams(needs_layout_passes=False),
        scratch_types=[
            pltpu.VMEM((pw + _CH,), jnp.float32),
            pltpu.VMEM((pw + _CH,), jnp.float32),
            pltpu.VMEM((pw + _CH,), jnp.float32),
            [pltpu.VMEM((8, _CH), jnp.int32) for _ in range(2)],
            [pltpu.VMEM((8 * _CH, 2 * _X), jnp.float32) for _ in range(2)],
            [pltpu.VMEM((12, _CH), jnp.float32) for _ in range(2)],
            [pltpu.VMEM((_CH,), jnp.int32) for _ in range(2)],
            pltpu.VMEM((pw,), jnp.float32),
            pltpu.SemaphoreType.DMA,
        ],
    )(table_rows, zc, yc, xc)


def kernel(idx, knots, depth):
    n = idx.shape[0]
    knots4 = knots.reshape(_D, _Z, _Y, _X)

    depths = jnp.arange(_D, dtype=jnp.float32)
    dv = jnp.asarray(depth).astype(jnp.float32)
    ind = jnp.searchsorted(depths, dv, side="right")
    norm = (dv - depths[ind - 1]) / (depths[ind] - depths[ind - 1])
    dloc = (ind - 1).astype(jnp.float32) + norm
    i0 = dloc.astype(jnp.int32)
    sd = jnp.clip(dloc - i0.astype(jnp.float32), 0.0, 1.0)
    idp = jnp.clip(i0 - 1 + jnp.arange(4, dtype=jnp.int32), 0, _D - 1)
    cvec = sd ** jnp.arange(3, -1, -1, dtype=jnp.float32)
    cd = cvec @ jnp.asarray(_CRB, dtype=jnp.float32)

    table = _collapse(idp, cd, knots4)
    table_rows = table.reshape(_NROW, 2 * _X)

    pw = ((n + 2 * _NW * _CH - 1) // (2 * _NW * _CH)) * 2 * _CH
    npad = _NW * pw
    zc = jnp.pad(idx[:, 0], (0, npad + _CH - n), constant_values=1.0)
    yc = jnp.pad(idx[:, 1], (0, npad + _CH - n), constant_values=1.0)
    xc = jnp.pad(idx[:, 2], (0, npad + _CH - n), constant_values=1.0)
    res = _interp(table_rows, zc, yc, xc, npad, pw)
    return res[:n].reshape(n, 1)


# speedup vs baseline: 1.2282x; 1.2282x over previous
import functools

import jax
import jax.numpy as jnp
import numpy as np
from jax import lax
from jax.experimental import pallas as pl
from jax.experimental.pallas import tpu as pltpu
from jax.experimental.pallas import tpu_sc as plsc

_D, _Z, _Y, _X = 64, 64, 64, 64
_NW = 32
_CH = 32
_G16 = _CH // 16
_NROW = 2 * _Z * (_Y // 2)

_H = np.array([[2., -2., 1., 1.], [-3., 3., -2., -1.],
               [0., 0., 1., 0.], [1., 0., 0., 0.]])
_CRB = _H @ np.array([[0., 1., 0., 0.], [0., 0., 1., 0.],
                      [-0.5, 0., 0.5, 0.], [0., -0.5, 0., 0.5]])


def _collapse_body(idp_ref, cd_ref, knots_ref, out_ref):
    d = pl.program_id(0)
    w = cd_ref[d]
    v = knots_ref[0]

    @pl.when(d == 0)
    def _():
        out_ref[...] = jnp.zeros_like(out_ref)

    out_ref[0] += w * v
    vr = jnp.concatenate([v[:, 1:, :], v[:, :1, :]], axis=1)
    out_ref[1] += w * vr


def _collapse(idp, cd, knots4):
    grid_spec = pltpu.PrefetchScalarGridSpec(
        num_scalar_prefetch=1,
        grid=(4,),
        in_specs=[
            pl.BlockSpec(memory_space=pltpu.SMEM),
            pl.BlockSpec((1, _Z, _Y, _X), lambda d, idp_ref: (idp_ref[d], 0, 0, 0)),
        ],
        out_specs=pl.BlockSpec((2, _Z, _Y, _X), lambda d, idp_ref: (0, 0, 0, 0)),
    )
    return pl.pallas_call(
        _collapse_body,
        grid_spec=grid_spec,
        out_shape=jax.ShapeDtypeStruct((2, _Z, _Y, _X), jnp.float32),
    )(idp, cd, knots4)


def _weights(s):
    w0 = ((-0.5 * s + 1.0) * s - 0.5) * s
    w1 = (1.5 * s - 2.5) * s * s + 1.0
    w2 = ((-1.5 * s + 2.0) * s + 0.5) * s
    w3 = (0.5 * s - 0.5) * s * s
    return w0, w1, w2, w3


def _axis_prep(coord, hi):
    i0 = coord.astype(jnp.int32)
    s = jnp.clip(coord - i0.astype(jnp.float32), 0.0, 1.0)
    b = jnp.clip(i0 - 1, 0, hi - 4)
    return b, _weights(s)


def _interp_body(pw, table, zc, yc, xc, out,
                 zbuf, ybuf, xbuf, idxbufs, valss, wbufs, xbbs, outbuf, sem):
    wid = lax.axis_index("s") * 2 + lax.axis_index("c")
    base = wid * pw
    pltpu.sync_copy(zc.at[pl.ds(base, pw + _CH)], zbuf)
    pltpu.sync_copy(yc.at[pl.ds(base, pw + _CH)], ybuf)
    pltpu.sync_copy(xc.at[pl.ds(base, pw + _CH)], xbuf)

    iota = lax.iota(jnp.int32, 16)

    def phase1(c, nb):
        idxbuf, wbuf, xbb = idxbufs[nb], wbufs[nb], xbbs[nb]
        for g in range(_G16):
            p0 = c * _CH + g * 16
            g16 = pl.ds(g * 16, 16)
            z = zbuf[pl.ds(p0, 16)]
            y = ybuf[pl.ds(p0, 16)]
            x = xbuf[pl.ds(p0, 16)]
            zb, wz = _axis_prep(z, _Z)
            yb, wy = _axis_prep(y, _Y)
            xb, wx = _axis_prep(x, _X)
            for k in range(4):
                wbuf[k, g16] = wz[k]
                wbuf[4 + k, g16] = wy[k]
                wbuf[8 + k, g16] = wx[k]
            xbb[g16] = xb
            ys = jnp.bitwise_and(yb, 1)
            k0 = lax.shift_right_logical(yb, 1)
            base0 = ys * (_Z * (_Y // 2)) + zb * (_Y // 2) + k0
            for a in range(4):
                for e in range(2):
                    idxbuf[a * 2 + e, g16] = base0 + (a * (_Y // 2) + e)
        for j in range(8):
            pltpu.async_copy(table.at[idxbuf.at[j]],
                             valss[nb].at[pl.ds(j * _CH, _CH)], sem)

    def drain(nb):
        for j in range(8):
            pltpu.make_async_copy(table.at[pl.ds(0, _CH)],
                                  valss[nb].at[pl.ds(j * _CH, _CH)], sem).wait()

    def phase2(c, nb):
        vals, wbuf, xbb = valss[nb], wbufs[nb], xbbs[nb]
        for g in range(_G16):
            g16 = pl.ds(g * 16, 16)
            pvec = iota + g * 16
            wz = [wbuf[k, g16] for k in range(4)]
            wy = [wbuf[4 + k, g16] for k in range(4)]
            wx = [wbuf[8 + k, g16] for k in range(4)]
            xb = xbb[g16]
            acc = jnp.zeros((16,), jnp.float32)
            for b in range(4):
                colv = xb + (b & 1) * 64
                for a in range(4):
                    rowv = pvec + (a * 2 + (b >> 1)) * _CH
                    t = plsc.load_gather(vals, [rowv, colv]) * wx[0]
                    t += plsc.load_gather(vals, [rowv, colv + 1]) * wx[1]
                    t += plsc.load_gather(vals, [rowv, colv + 2]) * wx[2]
                    t += plsc.load_gather(vals, [rowv, colv + 3]) * wx[3]
                    acc += t * (wz[a] * wy[b])
            outbuf[pl.ds(c * _CH + g * 16, 16)] = acc

    nch = pw // _CH
    phase1(0, 0)

    def pair(c2, carry):
        ca = 2 * c2
        phase1(ca + 1, 1)
        drain(0)
        phase2(ca, 0)
        phase1(ca + 2, 0)
        drain(1)
        phase2(ca + 1, 1)
        return carry

    lax.fori_loop(0, nch // 2, pair, 0)
    drain(0)
    pltpu.sync_copy(outbuf, out.at[pl.ds(base, pw)])


def _interp(table_rows, zc, yc, xc, npad, pw):
    mesh = plsc.VectorSubcoreMesh(core_axis_name="c", subcore_axis_name="s")
    return pl.kernel(
        functools.partial(_interp_body, pw),
        out_type=jax.ShapeDtypeStruct((npad,), jnp.float32),
        mesh=mesh,
        compiler_params=pltpu.CompilerParams(needs_layout_passes=False),
        scratch_types=[
            pltpu.VMEM((pw + _CH,), jnp.float32),
            pltpu.VMEM((pw + _CH,), jnp.float32),
            pltpu.VMEM((pw + _CH,), jnp.float32),
            [pltpu.VMEM((8, _CH), jnp.int32) for _ in range(2)],
            [pltpu.VMEM((8 * _CH, 2 * _X), jnp.float32) for _ in range(2)],
            [pltpu.VMEM((12, _CH), jnp.float32) for _ in range(2)],
            [pltpu.VMEM((_CH,), jnp.int32) for _ in range(2)],
            pltpu.VMEM((pw,), jnp.float32),
            pltpu.SemaphoreType.DMA,
        ],
    )(table_rows, zc, yc, xc)


def kernel(idx, knots, depth):
    n = idx.shape[0]
    knots4 = knots.reshape(_D, _Z, _Y, _X)

    depths = jnp.arange(_D, dtype=jnp.float32)
    dv = jnp.asarray(depth).astype(jnp.float32)
    ind = jnp.searchsorted(depths, dv, side="right")
    norm = (dv - depths[ind - 1]) / (depths[ind] - depths[ind - 1])
    dloc = (ind - 1).astype(jnp.float32) + norm
    i0 = dloc.astype(jnp.int32)
    sd = jnp.clip(dloc - i0.astype(jnp.float32), 0.0, 1.0)
    idp = jnp.clip(i0 - 1 + jnp.arange(4, dtype=jnp.int32), 0, _D - 1)
    cvec = sd ** jnp.arange(3, -1, -1, dtype=jnp.float32)
    cd = cvec @ jnp.asarray(_CRB, dtype=jnp.float32)

    table = _collapse(idp, cd, knots4)
    table_rows = table.reshape(_NROW, 2 * _X)

    pw = ((n + 2 * _NW * _CH - 1) // (2 * _NW * _CH)) * 2 * _CH
    npad = _NW * pw
    zc = jnp.pad(idx[:, 0], (0, npad + _CH - n), constant_values=1.0)
    yc = jnp.pad(idx[:, 1], (0, npad + _CH - n), constant_values=1.0)
    xc = jnp.pad(idx[:, 2], (0, npad + _CH - n), constant_values=1.0)
    res = _interp(table_rows, zc, yc, xc, npad, pw)
    return res[:n].reshape(n, 1)
